# Initial kernel scaffold; baseline (speedup 1.0000x reference)
import functools

import jax
import jax.numpy as jnp
from jax import lax
from jax.experimental import pallas as pl
from jax.experimental.pallas import tpu as pltpu
from jax.experimental.pallas import tpu_sc as plsc

N = 10000
E = 320000
IN_DIM = 128
HID = 64
LAT = 32
OUT_DIM = 128

NC = 2
NS = 16
NW = NC * NS
LANES = 16
CHUNK = 80
EPT = E // NW
RPS = N // NS


def _leaky(x):
    return jnp.where(x > 0, x, 0.2 * x)


def _pre1_body(x_ref, w1_ref, a1s_ref, a1d_ref,
               hext_ref, as_ref, ad_ref, off_ref):
    h = jnp.dot(x_ref[...], w1_ref[...], preferred_element_type=jnp.float32)
    asv = jnp.dot(h, a1s_ref[...][:, None],
                  preferred_element_type=jnp.float32)[:, 0]
    adv = jnp.dot(h, a1d_ref[...][:, None],
                  preferred_element_type=jnp.float32)[:, 0]
    m = jnp.max(asv)
    off = _leaky(m + adv)
    hext_ref[:, 0:HID] = h
    pad_iota = lax.broadcasted_iota(jnp.int32, (N, 16), 1)
    hext_ref[:, HID:HID + 16] = jnp.where(pad_iota == 0, 1.0, 0.0)
    as_ref[...] = asv
    ad_ref[...] = adv
    off_ref[...] = off


def _mid_body(part_ref, b1_ref, w2_ref, a2s_ref, a2d_ref,
              hext_ref, as_ref, ad_ref, off_ref):
    acc = part_ref[0] + part_ref[1]
    den = acc[:, HID]
    h1 = jnp.maximum(acc[:, 0:HID] / (den[:, None] + 1e-16) + b1_ref[...], 0.0)
    h2 = jnp.dot(h1, w2_ref[...], preferred_element_type=jnp.float32)
    asv = jnp.dot(h2, a2s_ref[...][:, None],
                  preferred_element_type=jnp.float32)[:, 0]
    adv = jnp.dot(h2, a2d_ref[...][:, None],
                  preferred_element_type=jnp.float32)[:, 0]
    m = jnp.max(asv)
    off = _leaky(m + adv)
    hext_ref[:, 0:LAT] = h2
    pad_iota = lax.broadcasted_iota(jnp.int32, (N, 16), 1)
    hext_ref[:, LAT:LAT + 16] = jnp.where(pad_iota == 0, 1.0, 0.0)
    as_ref[...] = asv
    ad_ref[...] = adv
    off_ref[...] = off


def _post_body(part_ref, b2_ref, wmu_ref, bmu_ref, wlv_ref, blv_ref,
               wd1_ref, bd1_ref, wd2_ref, bd2_ref, eps_ref,
               recon_ref, mu_ref, lv_ref):
    acc = part_ref[0] + part_ref[1]
    den = acc[:, LAT]
    h2 = jnp.maximum(acc[:, 0:LAT] / (den[:, None] + 1e-16) + b2_ref[...], 0.0)
    mu = jnp.dot(h2, wmu_ref[...], preferred_element_type=jnp.float32) + bmu_ref[...]
    lv = jnp.dot(h2, wlv_ref[...], preferred_element_type=jnp.float32) + blv_ref[...]
    z = mu + eps_ref[...] * jnp.exp(0.5 * lv)
    d = jnp.maximum(
        jnp.dot(z, wd1_ref[...], preferred_element_type=jnp.float32) + bd1_ref[...],
        0.0)
    recon_ref[...] = jax.nn.sigmoid(
        jnp.dot(d, wd2_ref[...], preferred_element_type=jnp.float32) + bd2_ref[...])
    mu_ref[...] = mu
    lv_ref[...] = lv


def _edge_sc(dp, hext, asv, adv, off, src, dst):
    gp = dp // LANES
    zrows = RPS // 5
    mesh = plsc.VectorSubcoreMesh(core_axis_name="c", subcore_axis_name="s")

    @functools.partial(
        pl.kernel,
        mesh=mesh,
        out_type=jax.ShapeDtypeStruct((NC, N, dp), jnp.float32),
        scratch_types=[
            pltpu.VMEM((N,), jnp.float32),
            pltpu.VMEM((N,), jnp.float32),
            pltpu.VMEM((N,), jnp.float32),
            pltpu.VMEM((CHUNK,), jnp.int32),
            pltpu.VMEM((CHUNK,), jnp.int32),
            pltpu.VMEM((CHUNK,), jnp.float32),
            pltpu.VMEM((CHUNK, dp), jnp.float32),
            pltpu.VMEM((zrows, dp), jnp.float32),
            pltpu.VMEM_SHARED((N, dp), jnp.float32),
            pltpu.SemaphoreType.DMA,
        ],
    )
    def k(hext_hbm, as_hbm, ad_hbm, off_hbm, src_hbm, dst_hbm, part_hbm,
          as_t, ad_t, off_t, sidx, didx, wbuf, rows, zbuf, num_sh, sem):
        c = lax.axis_index("c")
        s = lax.axis_index("s")
        wid = s * NC + c

        pltpu.sync_copy(as_hbm, as_t)
        pltpu.sync_copy(ad_hbm, ad_t)
        pltpu.sync_copy(off_hbm, off_t)

        def _z(i, _):
            for g in range(gp):
                zbuf[i, pl.ds(g * LANES, LANES)] = jnp.zeros((LANES,),
                                                             jnp.float32)
            return 0
        lax.fori_loop(0, zrows, _z, 0)
        for b in range(5):
            pltpu.sync_copy(zbuf, num_sh.at[pl.ds(s * RPS + b * zrows, zrows)])
        plsc.subcore_barrier()

        def chunk(kk, _):
            base = wid * EPT + kk * CHUNK
            pltpu.sync_copy(src_hbm.at[pl.ds(base, CHUNK)], sidx)
            pltpu.sync_copy(dst_hbm.at[pl.ds(base, CHUNK)], didx)
            pltpu.async_copy(hext_hbm.at[sidx], rows, sem).wait()
            for j in range(CHUNK // LANES):
                sv = sidx[pl.ds(j * LANES, LANES)]
                dv = didx[pl.ds(j * LANES, LANES)]
                a_s = plsc.load_gather(as_t, [sv])
                a_d = plsc.load_gather(ad_t, [dv])
                o_d = plsc.load_gather(off_t, [dv])
                e = a_s + a_d
                e = jnp.where(e > 0, e, 0.2 * e)
                wbuf[pl.ds(j * LANES, LANES)] = jnp.exp(e - o_d)

            def rowmul(r, _):
                w0 = wbuf[r]
                for g in range(gp):
                    rows[r, pl.ds(g * LANES, LANES)] = (
                        rows[r, pl.ds(g * LANES, LANES)] * w0)
                return 0
            lax.fori_loop(0, CHUNK, rowmul, 0)
            pltpu.sync_copy(rows, num_sh.at[didx], add=True)
            return 0

        lax.fori_loop(0, EPT // CHUNK, chunk, 0)
        plsc.subcore_barrier()
        pltpu.sync_copy(num_sh.at[pl.ds(s * RPS, RPS)],
                        part_hbm.at[c, pl.ds(s * RPS, RPS)])

    return k(hext, asv, adv, off, src, dst)


def kernel(x, edge_index, W1, a1_src, a1_dst, b1, W2, a2_src, a2_dst, b2,
           Wmu, bmu, Wlv, blv, Wd1, bd1, Wd2, bd2):
    src = edge_index[0]
    dst = edge_index[1]

    hext1, as1, ad1, off1 = pl.pallas_call(
        _pre1_body,
        out_shape=[
            jax.ShapeDtypeStruct((N, HID + 16), jnp.float32),
            jax.ShapeDtypeStruct((N,), jnp.float32),
            jax.ShapeDtypeStruct((N,), jnp.float32),
            jax.ShapeDtypeStruct((N,), jnp.float32),
        ],
    )(x, W1, a1_src, a1_dst)

    part1 = _edge_sc(HID + 16, hext1, as1, ad1, off1, src, dst)

    hext2, as2, ad2, off2 = pl.pallas_call(
        _mid_body,
        out_shape=[
            jax.ShapeDtypeStruct((N, LAT + 16), jnp.float32),
            jax.ShapeDtypeStruct((N,), jnp.float32),
            jax.ShapeDtypeStruct((N,), jnp.float32),
            jax.ShapeDtypeStruct((N,), jnp.float32),
        ],
    )(part1, b1, W2, a2_src, a2_dst)

    part2 = _edge_sc(LAT + 16, hext2, as2, ad2, off2, src, dst)

    eps = jax.random.normal(jax.random.key(42), (N, LAT), dtype=jnp.float32)
    recon, mu, logvar = pl.pallas_call(
        _post_body,
        out_shape=[
            jax.ShapeDtypeStruct((N, OUT_DIM), jnp.float32),
            jax.ShapeDtypeStruct((N, LAT), jnp.float32),
            jax.ShapeDtypeStruct((N, LAT), jnp.float32),
        ],
    )(part2, b2, Wmu, bmu, Wlv, blv, Wd1, bd1, Wd2, bd2, eps)

    return (recon, mu, logvar)


# speedup vs baseline: 28.7986x; 28.7986x over previous
import functools

import jax
import jax.numpy as jnp
from jax import lax
from jax.experimental import pallas as pl
from jax.experimental.pallas import tpu as pltpu
from jax.experimental.pallas import tpu_sc as plsc

N = 10000
E = 320000
IN_DIM = 128
HID = 64
LAT = 32
OUT_DIM = 128

NC = 2
NS = 16
NW = NC * NS
LANES = 16
CHUNK = 80
EPT = E // NW
NP = 10240
RPS = NP // NS


def _leaky(x):
    return jnp.where(x > 0, x, 0.2 * x)


def _pre1_body(x_ref, w1_ref, a1s_ref, a1d_ref,
               hext_ref, as_ref, ad_ref, off_ref):
    h = jnp.dot(x_ref[...], w1_ref[...], preferred_element_type=jnp.float32)
    asv = jnp.dot(h, a1s_ref[...][:, None],
                  preferred_element_type=jnp.float32)[:, 0]
    adv = jnp.dot(h, a1d_ref[...][:, None],
                  preferred_element_type=jnp.float32)[:, 0]
    m = jnp.max(asv)
    off = _leaky(m + adv)
    hext_ref[:, 0:HID] = h
    pad_iota = lax.broadcasted_iota(jnp.int32, (N, 16), 1)
    hext_ref[:, HID:HID + 16] = jnp.where(pad_iota == 0, 1.0, 0.0)
    as_ref[...] = asv
    ad_ref[...] = adv
    off_ref[...] = off


def _mid_body(part_ref, b1_ref, w2_ref, a2s_ref, a2d_ref,
              hext_ref, as_ref, ad_ref, off_ref):
    acc = part_ref[0, 0:N] + part_ref[1, 0:N]
    den = acc[:, HID]
    h1 = jnp.maximum(acc[:, 0:HID] / (den[:, None] + 1e-16) + b1_ref[...], 0.0)
    h2 = jnp.dot(h1, w2_ref[...], preferred_element_type=jnp.float32)
    asv = jnp.dot(h2, a2s_ref[...][:, None],
                  preferred_element_type=jnp.float32)[:, 0]
    adv = jnp.dot(h2, a2d_ref[...][:, None],
                  preferred_element_type=jnp.float32)[:, 0]
    m = jnp.max(asv)
    off = _leaky(m + adv)
    hext_ref[:, 0:LAT] = h2
    pad_iota = lax.broadcasted_iota(jnp.int32, (N, 16), 1)
    hext_ref[:, LAT:LAT + 16] = jnp.where(pad_iota == 0, 1.0, 0.0)
    as_ref[...] = asv
    ad_ref[...] = adv
    off_ref[...] = off


def _post_body(part_ref, b2_ref, wmu_ref, bmu_ref, wlv_ref, blv_ref,
               wd1_ref, bd1_ref, wd2_ref, bd2_ref, eps_ref,
               recon_ref, mu_ref, lv_ref):
    acc = part_ref[0, 0:N] + part_ref[1, 0:N]
    den = acc[:, LAT]
    h2 = jnp.maximum(acc[:, 0:LAT] / (den[:, None] + 1e-16) + b2_ref[...], 0.0)
    mu = jnp.dot(h2, wmu_ref[...], preferred_element_type=jnp.float32) + bmu_ref[...]
    lv = jnp.dot(h2, wlv_ref[...], preferred_element_type=jnp.float32) + blv_ref[...]
    z = mu + eps_ref[...] * jnp.exp(0.5 * lv)
    d = jnp.maximum(
        jnp.dot(z, wd1_ref[...], preferred_element_type=jnp.float32) + bd1_ref[...],
        0.0)
    recon_ref[...] = jax.nn.sigmoid(
        jnp.dot(d, wd2_ref[...], preferred_element_type=jnp.float32) + bd2_ref[...])
    mu_ref[...] = mu
    lv_ref[...] = lv


def _edge_sc(dp, hext, asv, adv, off, src, dst):
    gp = dp // LANES
    zrows = RPS // 5
    mesh = plsc.VectorSubcoreMesh(core_axis_name="c", subcore_axis_name="s")

    @functools.partial(
        pl.kernel,
        mesh=mesh,
        compiler_params=pltpu.CompilerParams(needs_layout_passes=False,
                                             use_tc_tiling_on_sc=False),
        out_type=jax.ShapeDtypeStruct((NC, NP, dp), jnp.float32),
        scratch_types=[
            pltpu.VMEM((N,), jnp.float32),
            pltpu.VMEM((N,), jnp.float32),
            pltpu.VMEM((N,), jnp.float32),
            pltpu.VMEM((CHUNK,), jnp.int32),
            pltpu.VMEM((CHUNK,), jnp.int32),
            pltpu.VMEM((CHUNK,), jnp.float32),
            pltpu.VMEM((CHUNK, dp), jnp.float32),
            pltpu.VMEM((zrows, dp), jnp.float32),
            pltpu.VMEM_SHARED((NP, dp), jnp.float32),
            pltpu.SemaphoreType.DMA,
        ],
    )
    def k(hext_hbm, as_hbm, ad_hbm, off_hbm, src_hbm, dst_hbm, part_hbm,
          as_t, ad_t, off_t, sidx, didx, wbuf, rows, zbuf, num_sh, sem):
        c = lax.axis_index("c")
        s = lax.axis_index("s")
        wid = s * NC + c

        pltpu.sync_copy(as_hbm, as_t)
        pltpu.sync_copy(ad_hbm, ad_t)
        pltpu.sync_copy(off_hbm, off_t)

        def _z(i, _):
            for g in range(gp):
                zbuf[i, pl.ds(g * LANES, LANES)] = jnp.zeros((LANES,),
                                                             jnp.float32)
            return 0
        lax.fori_loop(0, zrows, _z, 0)
        for b in range(5):
            pltpu.sync_copy(zbuf, num_sh.at[pl.ds(s * RPS + b * zrows, zrows)])
        plsc.subcore_barrier()

        def chunk(kk, _):
            base = wid * EPT + kk * CHUNK
            pltpu.sync_copy(src_hbm.at[pl.ds(base, CHUNK)], sidx)
            pltpu.sync_copy(dst_hbm.at[pl.ds(base, CHUNK)], didx)
            pltpu.async_copy(hext_hbm.at[sidx], rows, sem).wait()
            for j in range(CHUNK // LANES):
                sv = sidx[pl.ds(j * LANES, LANES)]
                dv = didx[pl.ds(j * LANES, LANES)]
                a_s = plsc.load_gather(as_t, [sv])
                a_d = plsc.load_gather(ad_t, [dv])
                o_d = plsc.load_gather(off_t, [dv])
                e = a_s + a_d
                e = jnp.where(e > 0, e, 0.2 * e)
                wbuf[pl.ds(j * LANES, LANES)] = jnp.exp(e - o_d)

            def rowblk(j, _):
                wv = wbuf[pl.ds(j * LANES, LANES)]
                for i in range(LANES):
                    w0 = wv[i]
                    r = j * LANES + i
                    for g in range(gp):
                        rows[r, pl.ds(g * LANES, LANES)] = (
                            rows[r, pl.ds(g * LANES, LANES)] * w0)
                return 0
            lax.fori_loop(0, CHUNK // LANES, rowblk, 0)
            pltpu.sync_copy(rows, num_sh.at[didx], add=True)
            return 0

        lax.fori_loop(0, EPT // CHUNK, chunk, 0)
        plsc.subcore_barrier()
        pltpu.sync_copy(num_sh.at[pl.ds(s * RPS, RPS)],
                        part_hbm.at[c, pl.ds(s * RPS, RPS)])

    return k(hext, asv, adv, off, src, dst)


def kernel(x, edge_index, W1, a1_src, a1_dst, b1, W2, a2_src, a2_dst, b2,
           Wmu, bmu, Wlv, blv, Wd1, bd1, Wd2, bd2):
    src = edge_index[0]
    dst = edge_index[1]

    tc_params = pltpu.CompilerParams(vmem_limit_bytes=110 * 1024 * 1024)
    hext1, as1, ad1, off1 = pl.pallas_call(
        _pre1_body,
        compiler_params=tc_params,
        out_shape=[
            jax.ShapeDtypeStruct((N, HID + 16), jnp.float32),
            jax.ShapeDtypeStruct((N,), jnp.float32),
            jax.ShapeDtypeStruct((N,), jnp.float32),
            jax.ShapeDtypeStruct((N,), jnp.float32),
        ],
    )(x, W1, a1_src, a1_dst)

    part1 = _edge_sc(HID + 16, hext1, as1, ad1, off1, src, dst)

    hext2, as2, ad2, off2 = pl.pallas_call(
        _mid_body,
        compiler_params=tc_params,
        out_shape=[
            jax.ShapeDtypeStruct((N, LAT + 16), jnp.float32),
            jax.ShapeDtypeStruct((N,), jnp.float32),
            jax.ShapeDtypeStruct((N,), jnp.float32),
            jax.ShapeDtypeStruct((N,), jnp.float32),
        ],
    )(part1, b1, W2, a2_src, a2_dst)

    part2 = _edge_sc(LAT + 16, hext2, as2, ad2, off2, src, dst)

    eps = jax.random.normal(jax.random.key(42), (N, LAT), dtype=jnp.float32)
    recon, mu, logvar = pl.pallas_call(
        _post_body,
        compiler_params=tc_params,
        out_shape=[
            jax.ShapeDtypeStruct((N, OUT_DIM), jnp.float32),
            jax.ShapeDtypeStruct((N, LAT), jnp.float32),
            jax.ShapeDtypeStruct((N, LAT), jnp.float32),
        ],
    )(part2, b2, Wmu, bmu, Wlv, blv, Wd1, bd1, Wd2, bd2, eps)

    return (recon, mu, logvar)


# speedup vs baseline: 60.1392x; 2.0883x over previous
import functools

import jax
import jax.numpy as jnp
from jax import lax
from jax.experimental import pallas as pl
from jax.experimental.pallas import tpu as pltpu
from jax.experimental.pallas import tpu_sc as plsc

N = 10000
E = 320000
IN_DIM = 128
HID = 64
LAT = 32
OUT_DIM = 128

NC = 2
NS = 16
NW = NC * NS
LANES = 16
CHUNK = 80
EPT = E // NW
NP = 10240
RPS = NP // NS


def _leaky(x):
    return jnp.where(x > 0, x, 0.2 * x)


def _pre1_body(x_ref, w1_ref, a1s_ref, a1d_ref,
               hext_ref, as_ref, ad_ref, off_ref):
    h = jnp.dot(x_ref[...], w1_ref[...], preferred_element_type=jnp.float32)
    asv = jnp.dot(h, a1s_ref[...][:, None],
                  preferred_element_type=jnp.float32)[:, 0]
    adv = jnp.dot(h, a1d_ref[...][:, None],
                  preferred_element_type=jnp.float32)[:, 0]
    m = jnp.max(asv)
    off = _leaky(m + adv)
    hext_ref[:, 0:HID] = h
    pad_iota = lax.broadcasted_iota(jnp.int32, (N, 16), 1)
    hext_ref[:, HID:HID + 16] = jnp.where(pad_iota == 0, 1.0, 0.0)
    as_ref[...] = asv
    ad_ref[...] = adv
    off_ref[...] = off


def _mid_body(part_ref, b1_ref, w2_ref, a2s_ref, a2d_ref,
              hext_ref, as_ref, ad_ref, off_ref):
    acc = part_ref[0, 0:N] + part_ref[1, 0:N]
    den = acc[:, HID]
    h1 = jnp.maximum(acc[:, 0:HID] / (den[:, None] + 1e-16) + b1_ref[...], 0.0)
    h2 = jnp.dot(h1, w2_ref[...], preferred_element_type=jnp.float32)
    asv = jnp.dot(h2, a2s_ref[...][:, None],
                  preferred_element_type=jnp.float32)[:, 0]
    adv = jnp.dot(h2, a2d_ref[...][:, None],
                  preferred_element_type=jnp.float32)[:, 0]
    m = jnp.max(asv)
    off = _leaky(m + adv)
    hext_ref[:, 0:LAT] = h2
    pad_iota = lax.broadcasted_iota(jnp.int32, (N, 16), 1)
    hext_ref[:, LAT:LAT + 16] = jnp.where(pad_iota == 0, 1.0, 0.0)
    as_ref[...] = asv
    ad_ref[...] = adv
    off_ref[...] = off


def _post_body(part_ref, b2_ref, wmu_ref, bmu_ref, wlv_ref, blv_ref,
               wd1_ref, bd1_ref, wd2_ref, bd2_ref, eps_ref,
               recon_ref, mu_ref, lv_ref):
    acc = part_ref[0, 0:N] + part_ref[1, 0:N]
    den = acc[:, LAT]
    h2 = jnp.maximum(acc[:, 0:LAT] / (den[:, None] + 1e-16) + b2_ref[...], 0.0)
    mu = jnp.dot(h2, wmu_ref[...], preferred_element_type=jnp.float32) + bmu_ref[...]
    lv = jnp.dot(h2, wlv_ref[...], preferred_element_type=jnp.float32) + blv_ref[...]
    z = mu + eps_ref[...] * jnp.exp(0.5 * lv)
    d = jnp.maximum(
        jnp.dot(z, wd1_ref[...], preferred_element_type=jnp.float32) + bd1_ref[...],
        0.0)
    recon_ref[...] = jax.nn.sigmoid(
        jnp.dot(d, wd2_ref[...], preferred_element_type=jnp.float32) + bd2_ref[...])
    mu_ref[...] = mu
    lv_ref[...] = lv


def _edge_sc(dp, hext, asv, adv, off, src, dst):
    gp = dp // LANES
    zrows = RPS // 5
    mesh = plsc.VectorSubcoreMesh(core_axis_name="c", subcore_axis_name="s")

    nchunks = EPT // CHUNK

    @functools.partial(
        pl.kernel,
        mesh=mesh,
        compiler_params=pltpu.CompilerParams(needs_layout_passes=False,
                                             use_tc_tiling_on_sc=False),
        out_type=jax.ShapeDtypeStruct((NC, NP, dp), jnp.float32),
        scratch_types=[
            pltpu.VMEM((N,), jnp.float32),
            pltpu.VMEM((N,), jnp.float32),
            pltpu.VMEM((N,), jnp.float32),
            pltpu.VMEM((nchunks, CHUNK), jnp.int32),
            pltpu.VMEM((nchunks, CHUNK), jnp.int32),
            pltpu.VMEM((CHUNK, dp), jnp.float32),
            pltpu.VMEM((CHUNK, dp), jnp.float32),
            pltpu.VMEM((zrows, dp), jnp.float32),
            pltpu.VMEM_SHARED((NP, dp), jnp.float32),
            pltpu.SemaphoreType.DMA,
            pltpu.SemaphoreType.DMA,
        ],
    )
    def k(hext_hbm, as_hbm, ad_hbm, off_hbm, src_hbm, dst_hbm, part_hbm,
          as_t, ad_t, off_t, sidx, didx, rows0, rows1, zbuf, num_sh,
          gsem0, gsem1):
        c = lax.axis_index("c")
        s = lax.axis_index("s")
        wid = s * NC + c
        rows = (rows0, rows1)
        gsem = (gsem0, gsem1)

        pltpu.sync_copy(src_hbm.at[pl.ds(wid * nchunks, nchunks)], sidx)
        pltpu.sync_copy(dst_hbm.at[pl.ds(wid * nchunks, nchunks)], didx)
        pltpu.sync_copy(as_hbm, as_t)
        pltpu.sync_copy(ad_hbm, ad_t)
        pltpu.sync_copy(off_hbm, off_t)

        def _z(i, _):
            for g in range(gp):
                zbuf[i, pl.ds(g * LANES, LANES)] = jnp.zeros((LANES,),
                                                             jnp.float32)
            return 0
        lax.fori_loop(0, zrows, _z, 0)
        for b in range(5):
            pltpu.sync_copy(zbuf, num_sh.at[pl.ds(s * RPS + b * zrows, zrows)])
        plsc.subcore_barrier()

        def process(kk, b):
            rb = rows[b]
            for j in range(CHUNK // LANES):
                sv = sidx[kk, pl.ds(j * LANES, LANES)]
                dv = didx[kk, pl.ds(j * LANES, LANES)]
                a_s = plsc.load_gather(as_t, [sv])
                a_d = plsc.load_gather(ad_t, [dv])
                o_d = plsc.load_gather(off_t, [dv])
                e = a_s + a_d
                e = jnp.where(e > 0, e, 0.2 * e)
                wv = jnp.exp(e - o_d)
                for i in range(LANES):
                    w0 = wv[i]
                    r = j * LANES + i
                    for g in range(gp):
                        rb[r, pl.ds(g * LANES, LANES)] = (
                            rb[r, pl.ds(g * LANES, LANES)] * w0)
            pltpu.sync_copy(rb, num_sh.at[didx.at[kk]], add=True)

        def start_gather(kk, b):
            return pltpu.async_copy(hext_hbm.at[sidx.at[kk]], rows[b], gsem[b])

        g0 = start_gather(0, 0)

        def pair(p, _):
            for b in range(2):
                kk = 2 * p + b
                start_gather(kk + 1, 1 - b)
                pltpu.make_async_copy(hext_hbm.at[sidx.at[kk]],
                                      rows[b], gsem[b]).wait()
                process(kk, b)
            return 0

        lax.fori_loop(0, (nchunks - 1) // 2, pair, 0)
        last = nchunks - 1
        pltpu.make_async_copy(hext_hbm.at[sidx.at[last]],
                              rows[last % 2], gsem[last % 2]).wait()
        process(last, last % 2)

        plsc.subcore_barrier()
        pltpu.sync_copy(num_sh.at[pl.ds(s * RPS, RPS)],
                        part_hbm.at[c, pl.ds(s * RPS, RPS)])

    src2 = src.reshape(NW * nchunks, CHUNK)
    dst2 = dst.reshape(NW * nchunks, CHUNK)
    return k(hext, asv, adv, off, src2, dst2)


def kernel(x, edge_index, W1, a1_src, a1_dst, b1, W2, a2_src, a2_dst, b2,
           Wmu, bmu, Wlv, blv, Wd1, bd1, Wd2, bd2):
    src = edge_index[0]
    dst = edge_index[1]

    tc_params = pltpu.CompilerParams(vmem_limit_bytes=110 * 1024 * 1024)
    hext1, as1, ad1, off1 = pl.pallas_call(
        _pre1_body,
        compiler_params=tc_params,
        out_shape=[
            jax.ShapeDtypeStruct((N, HID + 16), jnp.float32),
            jax.ShapeDtypeStruct((N,), jnp.float32),
            jax.ShapeDtypeStruct((N,), jnp.float32),
            jax.ShapeDtypeStruct((N,), jnp.float32),
        ],
    )(x, W1, a1_src, a1_dst)

    part1 = _edge_sc(HID + 16, hext1, as1, ad1, off1, src, dst)

    hext2, as2, ad2, off2 = pl.pallas_call(
        _mid_body,
        compiler_params=tc_params,
        out_shape=[
            jax.ShapeDtypeStruct((N, LAT + 16), jnp.float32),
            jax.ShapeDtypeStruct((N,), jnp.float32),
            jax.ShapeDtypeStruct((N,), jnp.float32),
            jax.ShapeDtypeStruct((N,), jnp.float32),
        ],
    )(part1, b1, W2, a2_src, a2_dst)

    part2 = _edge_sc(LAT + 16, hext2, as2, ad2, off2, src, dst)

    eps = jax.random.normal(jax.random.key(42), (N, LAT), dtype=jnp.float32)
    recon, mu, logvar = pl.pallas_call(
        _post_body,
        compiler_params=tc_params,
        out_shape=[
            jax.ShapeDtypeStruct((N, OUT_DIM), jnp.float32),
            jax.ShapeDtypeStruct((N, LAT), jnp.float32),
            jax.ShapeDtypeStruct((N, LAT), jnp.float32),
        ],
    )(part2, b2, Wmu, bmu, Wlv, blv, Wd1, bd1, Wd2, bd2, eps)

    return (recon, mu, logvar)
